# baseline (speedup 1.0000x reference)
import functools

import jax
import jax.numpy as jnp
from jax import lax
from jax.experimental import pallas as pl
from jax.experimental.pallas import tpu as pltpu
from jax.experimental.pallas import tpu_sc as plsc

D_MODEL = 1024
BATCH = 4
SEQ = 2048
B = BATCH * SEQ
NC, NS, L = 2, 16, 16
NW = NC * NS
BPW = B // NW
CH = 32
NCHUNK = BPW // CH
SCALE = 32.0

_mesh = plsc.VectorSubcoreMesh(core_axis_name="c", subcore_axis_name="s")


WPB = SEQ // BPW


@functools.partial(
    pl.kernel,
    mesh=_mesh,
    out_type=jax.ShapeDtypeStruct((BATCH, SEQ, D_MODEL), jnp.float32),
    scratch_types=[
        pltpu.VMEM((BPW,), jnp.int32),
        pltpu.VMEM((CH, D_MODEL), jnp.float32),
        pltpu.VMEM((CH, D_MODEL), jnp.float32),
        pltpu.SemaphoreType.DMA,
        pltpu.SemaphoreType.DMA,
        pltpu.SemaphoreType.DMA,
        pltpu.SemaphoreType.DMA,
    ],
)
def _emb_lookup(x_hbm, table_hbm, out_hbm, idx_v, rows0, rows1,
                g0, g1, o0, o1):
    wid = lax.axis_index("s") * NC + lax.axis_index("c")
    bi = wid // WPB
    s0 = (wid % WPB) * BPW
    pltpu.sync_copy(x_hbm.at[bi, pl.ds(s0, BPW)], idx_v)

    def _scale_buf(buf):
        def srow(r, _):
            def scol(j, _):
                buf[r, pl.ds(j * L, L)] = buf[r, pl.ds(j * L, L)] * SCALE
                return 0
            return lax.fori_loop(0, D_MODEL // L, scol, 0, unroll=8)
        lax.fori_loop(0, CH, srow, 0)

    def _gather(c, buf, sem):
        return pltpu.async_copy(table_hbm.at[idx_v.at[pl.ds(c * CH, CH)]],
                                buf, sem)

    def _drain(buf, sem):
        pltpu.make_async_copy(table_hbm.at[pl.ds(0, CH)], buf, sem).wait()

    def _scatter(c, buf, sem):
        return pltpu.async_copy(buf, out_hbm.at[bi, pl.ds(s0 + c * CH, CH)],
                                sem)

    T = NCHUNK // 2
    _gather(0, rows0, g0)

    def body(t, _):
        c0 = 2 * t
        _drain(rows0, g0)
        @pl.when(t > 0)
        def _():
            _drain(rows1, o1)
        _gather(c0 + 1, rows1, g1)
        _scale_buf(rows0)
        _scatter(c0, rows0, o0)
        _drain(rows1, g1)
        _scale_buf(rows1)
        @pl.when(t < T - 1)
        def _():
            _drain(rows0, o0)
            _gather(c0 + 2, rows0, g0)
        _scatter(c0 + 1, rows1, o1)
        return 0

    lax.fori_loop(0, T, body, 0)
    _drain(rows0, o0)
    _drain(rows1, o1)


def kernel(x, table):
    return _emb_lookup(x, table)


# speedup vs baseline: 1.0388x; 1.0388x over previous
import functools

import jax
import jax.numpy as jnp
from jax import lax
from jax.experimental import pallas as pl
from jax.experimental.pallas import tpu as pltpu
from jax.experimental.pallas import tpu_sc as plsc

D_MODEL = 1024
BATCH = 4
SEQ = 2048
B = BATCH * SEQ
NC, NS, L = 2, 16, 16
NW = NC * NS
BPW = B // NW
CH = 32
NCHUNK = BPW // CH
NBUF = 3
SCALE = 32.0

_mesh = plsc.VectorSubcoreMesh(core_axis_name="c", subcore_axis_name="s")


WPB = SEQ // BPW


@functools.partial(
    pl.kernel,
    mesh=_mesh,
    out_type=jax.ShapeDtypeStruct((BATCH, SEQ, D_MODEL), jnp.float32),
    scratch_types=[
        pltpu.VMEM((BPW,), jnp.int32),
        pltpu.VMEM((CH, D_MODEL), jnp.float32),
        pltpu.VMEM((CH, D_MODEL), jnp.float32),
        pltpu.VMEM((CH, D_MODEL), jnp.float32),
        pltpu.SemaphoreType.DMA,
        pltpu.SemaphoreType.DMA,
        pltpu.SemaphoreType.DMA,
        pltpu.SemaphoreType.DMA,
        pltpu.SemaphoreType.DMA,
        pltpu.SemaphoreType.DMA,
    ],
)
def _emb_lookup(x_hbm, table_hbm, out_hbm, idx_v, rows0, rows1, rows2,
                g0, g1, g2, o0, o1, o2):
    wid = lax.axis_index("s") * NC + lax.axis_index("c")
    bi = wid // WPB
    s0 = (wid % WPB) * BPW
    pltpu.sync_copy(x_hbm.at[bi, pl.ds(s0, BPW)], idx_v)

    bufs = (rows0, rows1, rows2)
    gsems = (g0, g1, g2)
    osems = (o0, o1, o2)
    gathers = [None] * NBUF
    outs = [None] * NBUF

    def _scale_buf(buf):
        def srow(r, _):
            def scol(j, _):
                buf[r, pl.ds(j * L, L)] = buf[r, pl.ds(j * L, L)] * SCALE
                return 0
            return lax.fori_loop(0, D_MODEL // L, scol, 0, unroll=8)
        lax.fori_loop(0, CH, srow, 0)

    def _issue_gather(c):
        b = c % NBUF
        if outs[b] is not None:
            outs[b].wait()
            outs[b] = None
        gathers[b] = pltpu.async_copy(
            table_hbm.at[idx_v.at[pl.ds(c * CH, CH)]], bufs[b], gsems[b])

    for c in range(NBUF - 1):
        _issue_gather(c)

    for c in range(NCHUNK):
        b = c % NBUF
        if c + NBUF - 1 < NCHUNK:
            _issue_gather(c + NBUF - 1)
        gathers[b].wait()
        _scale_buf(bufs[b])
        outs[b] = pltpu.async_copy(
            bufs[b], out_hbm.at[bi, pl.ds(s0 + c * CH, CH)], osems[b])

    for b in range(NBUF):
        if outs[b] is not None:
            outs[b].wait()


def kernel(x, table):
    return _emb_lookup(x, table)


# speedup vs baseline: 1.0732x; 1.0331x over previous
import functools

import jax
import jax.numpy as jnp
from jax import lax
from jax.experimental import pallas as pl
from jax.experimental.pallas import tpu as pltpu
from jax.experimental.pallas import tpu_sc as plsc

D_MODEL = 1024
BATCH = 4
SEQ = 2048
B = BATCH * SEQ
NC, NS, L = 2, 16, 16
NW = NC * NS
BPW = B // NW
CH = 16
NCHUNK = BPW // CH
NBUF = 6
SCALE = 32.0

_mesh = plsc.VectorSubcoreMesh(core_axis_name="c", subcore_axis_name="s")


WPB = SEQ // BPW


@functools.partial(
    pl.kernel,
    mesh=_mesh,
    out_type=jax.ShapeDtypeStruct((BATCH, SEQ, D_MODEL), jnp.float32),
    scratch_types=(
        [pltpu.VMEM((BPW,), jnp.int32)]
        + [pltpu.VMEM((CH, D_MODEL), jnp.float32)] * NBUF
        + [pltpu.SemaphoreType.DMA] * (2 * NBUF)
    ),
)
def _emb_lookup(x_hbm, table_hbm, out_hbm, idx_v, *scratch):
    wid = lax.axis_index("s") * NC + lax.axis_index("c")
    bi = wid // WPB
    s0 = (wid % WPB) * BPW
    pltpu.sync_copy(x_hbm.at[bi, pl.ds(s0, BPW)], idx_v)

    bufs = scratch[:NBUF]
    gsems = scratch[NBUF:2 * NBUF]
    osems = scratch[2 * NBUF:]
    gathers = [None] * NBUF
    outs = [None] * NBUF

    def _scale_buf(buf):
        def srow(r, _):
            def scol(j, _):
                buf[r, pl.ds(j * L, L)] = buf[r, pl.ds(j * L, L)] * SCALE
                return 0
            return lax.fori_loop(0, D_MODEL // L, scol, 0, unroll=8)
        lax.fori_loop(0, CH, srow, 0)

    def _issue_gather(c):
        b = c % NBUF
        if outs[b] is not None:
            outs[b].wait()
            outs[b] = None
        gathers[b] = pltpu.async_copy(
            table_hbm.at[idx_v.at[pl.ds(c * CH, CH)]], bufs[b], gsems[b])

    for c in range(NBUF - 1):
        _issue_gather(c)

    for c in range(NCHUNK):
        b = c % NBUF
        if c + NBUF - 1 < NCHUNK:
            _issue_gather(c + NBUF - 1)
        gathers[b].wait()
        _scale_buf(bufs[b])
        outs[b] = pltpu.async_copy(
            bufs[b], out_hbm.at[bi, pl.ds(s0 + c * CH, CH)], osems[b])

    for b in range(NBUF):
        if outs[b] is not None:
            outs[b].wait()


def kernel(x, table):
    return _emb_lookup(x, table)
